# baseline (speedup 1.0000x reference)
import functools

import jax
import jax.numpy as jnp
from jax import lax
from jax.experimental import pallas as pl
from jax.experimental.pallas import tpu as pltpu
from jax.experimental.pallas import tpu_sc as plsc

_EMB = 64
_CHUNK = 128
_NW = 32
_NBUF = 2
_LANES = 16


def _gather_body(idx_hbm, tab_hbm, p_hbm, idx_v, q_v, h_v, pair_v, v_v,
                 sem_g, sem_w):
    n_chunks = idx_v.shape[0]
    wid = lax.axis_index("s") * 2 + lax.axis_index("c")
    cbase = wid * n_chunks
    nb = p_hbm.shape[2]

    pltpu.sync_copy(idx_hbm.at[pl.ds(cbase, n_chunks)], idx_v)

    iota = lax.iota(jnp.int32, _LANES)

    def stage(b, j):
        for k in range(_CHUNK // _LANES):
            iv = idx_v[j, pl.ds(k * _LANES, _LANES)]
            q_v[b, pl.ds(k * _LANES, _LANES)] = lax.shift_right_logical(iv, 1)
            h_v[b, pl.ds(k * _LANES, _LANES)] = (iv & 1) * _EMB
        pltpu.make_async_copy(
            tab_hbm.at[q_v.at[b]], pair_v.at[b, :, pl.ds(0, 2 * _EMB)],
            sem_g.at[b]).start()

    rows_k = [k * _LANES + iota for k in range(_CHUNK // _LANES)]

    def select(b):
        for k in range(_CHUNK // _LANES):
            h = h_v[b, pl.ds(k * _LANES, _LANES)]
            for d in range(_EMB):
                v_v[b, d, pl.ds(k * _LANES, _LANES)] = plsc.load_gather(
                    pair_v.at[b], [rows_k[k], h + d])

    def write(b, j):
        c = cbase + j
        f = c // 128
        b0 = (c % 128) * _CHUNK
        return pltpu.make_async_copy(
            v_v.at[b], p_hbm.at[f, :, pl.ds(b0, _CHUNK)], sem_w.at[b])

    def gwait(b):
        pltpu.make_async_copy(
            tab_hbm.at[q_v.at[b]], pair_v.at[b, :, pl.ds(0, 2 * _EMB)],
            sem_g.at[b]).wait()

    for b in range(_NBUF):
        stage(b, b)

    n_groups = n_chunks // _NBUF

    def group(g, carry):
        for b in range(_NBUF):
            j = g * _NBUF + b
            gwait(b)

            @pl.when(g > 0)
            def _():
                write(b, j - _NBUF).wait()

            select(b)
            write(b, j).start()

            @pl.when(g < n_groups - 1)
            def _():
                stage(b, j + _NBUF)
        return carry

    lax.fori_loop(0, n_groups, group, 0)

    for b in range(_NBUF):
        write(b, (n_groups - 1) * _NBUF + b).wait()


def kernel(idx, table):
    bsz, nf = idx.shape
    n_rows, emb = table.shape
    idx_c = jnp.transpose(idx).reshape(bsz * nf // _CHUNK, _CHUNK).astype(
        jnp.int32)
    tab2 = table.reshape(n_rows // 2, 2 * emb)
    n_chunks_w = idx_c.shape[0] // _NW
    mesh = plsc.VectorSubcoreMesh(core_axis_name="c", subcore_axis_name="s")
    run = functools.partial(
        pl.kernel,
        mesh=mesh,
        compiler_params=pltpu.CompilerParams(
            use_tc_tiling_on_sc=True, needs_layout_passes=False),
        out_type=jax.ShapeDtypeStruct((nf, emb, bsz), jnp.float32),
        scratch_types=[
            pltpu.VMEM((n_chunks_w, _CHUNK), jnp.int32),
            pltpu.VMEM((_NBUF, _CHUNK), jnp.int32),
            pltpu.VMEM((_NBUF, _CHUNK), jnp.int32),
            pltpu.VMEM((_NBUF, _CHUNK, 2 * emb + 8), jnp.float32),
            pltpu.VMEM((_NBUF, emb, _CHUNK), jnp.float32),
            pltpu.SemaphoreType.DMA((_NBUF,)),
            pltpu.SemaphoreType.DMA((_NBUF,)),
        ],
    )(_gather_body)
    p = run(idx_c, tab2)
    return jnp.transpose(p, (2, 0, 1))


# speedup vs baseline: 1.3783x; 1.3783x over previous
import functools

import jax
import jax.numpy as jnp
from jax import lax
from jax.experimental import pallas as pl
from jax.experimental.pallas import tpu as pltpu
from jax.experimental.pallas import tpu_sc as plsc

_EMB = 64
_CHUNK = 128
_NW = 32
_NBUF = 4


def _gather_body(idx_hbm, table_hbm, out_hbm, idx_v, buf_v, sem_g, sem_s):
    n_chunks = idx_v.shape[0]
    wid = lax.axis_index("s") * 2 + lax.axis_index("c")
    cbase = wid * n_chunks

    pltpu.sync_copy(idx_hbm.at[pl.ds(cbase, n_chunks)], idx_v)

    def gather(b, j):
        return pltpu.make_async_copy(
            table_hbm.at[idx_v.at[j]], buf_v.at[b], sem_g.at[b])

    def write(b, j):
        return pltpu.make_async_copy(
            buf_v.at[b],
            out_hbm.at[pl.ds((cbase + j) * _CHUNK, _CHUNK)],
            sem_s.at[b])

    for b in range(_NBUF):
        gather(b, b).start()

    n_groups = n_chunks // _NBUF

    def group(g, carry):
        for b in range(_NBUF):
            j = g * _NBUF + b
            gather(b, j).wait()
            write(b, j).start()
            write(b, j).wait()
            gather(b, j + _NBUF).start()
        return carry

    lax.fori_loop(0, n_groups - 1, group, 0)

    for b in range(_NBUF):
        j = (n_groups - 1) * _NBUF + b
        gather(b, j).wait()
        write(b, j).start()
        write(b, j).wait()


def kernel(idx, table):
    bsz = idx.shape[0] * idx.shape[1]
    idx2d = idx.reshape(bsz // _CHUNK, _CHUNK).astype(jnp.int32)
    n_chunks_w = idx2d.shape[0] // _NW
    mesh = plsc.VectorSubcoreMesh(core_axis_name="c", subcore_axis_name="s")
    run = functools.partial(
        pl.kernel,
        mesh=mesh,
        compiler_params=pltpu.CompilerParams(use_tc_tiling_on_sc=False),
        out_type=jax.ShapeDtypeStruct((bsz, _EMB), jnp.float32),
        scratch_types=[
            pltpu.VMEM((n_chunks_w, _CHUNK), jnp.int32),
            pltpu.VMEM((_NBUF, _CHUNK, _EMB), jnp.float32),
            pltpu.SemaphoreType.DMA((_NBUF,)),
            pltpu.SemaphoreType.DMA((_NBUF,)),
        ],
    )(_gather_body)
    out = run(idx2d, table)
    return out.reshape(idx.shape[0], idx.shape[1], _EMB)
